# baseline (speedup 1.0000x reference)
import functools
import jax
import jax.numpy as jnp
from jax import lax
from jax.experimental import pallas as pl
from jax.experimental.pallas import tpu as pltpu
from jax.experimental.pallas import tpu_sc as plsc

_E, _B, _D1, _D2 = 32, 32, 256, 256
_NW = 32
_RW = _D1 // _NW
_RC = 4
_NCH = _RW // _RC
_L = 16
_CV = _D2 // _L

_mesh = plsc.VectorSubcoreMesh(core_axis_name="c", subcore_axis_name="s")


def _sc_body(wT_hbm, W_hbm, out_hbm, wT_v, slab_a, slab_b, out_v, sem_a, sem_b):
    wid = lax.axis_index("s") * 2 + lax.axis_index("c")
    row0 = wid * _RW
    pltpu.sync_copy(wT_hbm, wT_v)

    slabs = (slab_a, slab_b)
    sems = (sem_a, sem_b)
    copies = []
    for ci in range(_NCH):
        copies.append(pltpu.async_copy(
            W_hbm.at[:, pl.ds(row0 + ci * _RC, _RC), :], slabs[ci], sems[ci]))

    for ci in range(_NCH):
        slab = slabs[ci]
        copies[ci].wait()

        def b_body(b2, _, slab=slab):
            b0 = b2 * 2
            b1 = b0 + 1
            wa0 = wT_v[b0, pl.ds(0, _L)]
            wa1 = wT_v[b0, pl.ds(_L, _L)]
            wb0 = wT_v[b1, pl.ds(0, _L)]
            wb1 = wT_v[b1, pl.ds(_L, _L)]
            sa = [wa0[i] for i in range(_L)] + [wa1[i] for i in range(_L)]
            sb = [wb0[i] for i in range(_L)] + [wb1[i] for i in range(_L)]

            def c_body(c, _):
                co = c * _L
                acc_a = [None] * _RC
                acc_b = [None] * _RC
                for r in range(_RC):
                    x = slab[0, r, pl.ds(co, _L)]
                    acc_a[r] = sa[0] * x
                    acc_b[r] = sb[0] * x
                for e in range(1, _E):
                    for r in range(_RC):
                        x = slab[e, r, pl.ds(co, _L)]
                        acc_a[r] = acc_a[r] + sa[e] * x
                        acc_b[r] = acc_b[r] + sb[e] * x
                for r in range(_RC):
                    out_v[b0, r, pl.ds(co, _L)] = acc_a[r]
                    out_v[b1, r, pl.ds(co, _L)] = acc_b[r]
                return 0

            lax.fori_loop(0, _CV, c_body, 0, unroll=2)
            return 0

        lax.fori_loop(0, _B // 2, b_body, 0)
        pltpu.sync_copy(out_v, out_hbm.at[:, pl.ds(row0 + ci * _RC, _RC), :])


def kernel(superposition_weights, W):
    k = pl.kernel(
        _sc_body,
        out_type=jax.ShapeDtypeStruct((_B, _D1, _D2), jnp.float32),
        mesh=_mesh,
        scratch_types=[
            pltpu.VMEM((_B, _E), jnp.float32),
            pltpu.VMEM((_E, _RC, _D2), jnp.float32),
            pltpu.VMEM((_E, _RC, _D2), jnp.float32),
            pltpu.VMEM((_B, _RC, _D2), jnp.float32),
            pltpu.SemaphoreType.DMA,
            pltpu.SemaphoreType.DMA,
        ],
    )
    return k(superposition_weights.T, W)


# speedup vs baseline: 12.1417x; 12.1417x over previous
import jax
import jax.numpy as jnp
from jax.experimental import pallas as pl

_RBLK = 128


def _body(w_ref, x_ref, o_ref):
    o_ref[...] = jax.lax.dot_general(
        w_ref[...], x_ref[...],
        dimension_numbers=(((0,), (0,)), ((), ())),
        preferred_element_type=jnp.float32,
    )


def kernel(superposition_weights, W):
    E, B = superposition_weights.shape
    _, d1, d2 = W.shape
    out = pl.pallas_call(
        _body,
        grid=(d1 // _RBLK,),
        in_specs=[
            pl.BlockSpec((E, B), lambda i: (0, 0)),
            pl.BlockSpec((E, _RBLK, d2), lambda i: (0, i, 0)),
        ],
        out_specs=pl.BlockSpec((B, _RBLK, d2), lambda i: (0, i, 0)),
        out_shape=jax.ShapeDtypeStruct((B, d1, d2), jnp.float32),
    )(superposition_weights, W)
    return out
